# baseline (speedup 1.0000x reference)
import functools

import jax
import jax.numpy as jnp
from jax import lax
from jax.experimental import pallas as pl
from jax.experimental.pallas import tpu as pltpu
from jax.experimental.pallas import tpu_sc as plsc

_CAP = 16384
_B = 4096
_D = 3 * 32 * 32

_NC = 2
_NS = 16
_NW = _NC * _NS
_RPW = _CAP // _NW
_NV = _B // 16
_BR = 128
_CH = 16


def _sc_body(bx_hbm, by_hbm, bt_hbm, x_hbm, y_hbm, t_hbm, idx_hbm,
             obx_hbm, oby_hbm, obt_hbm,
             idx_v, last_v, srcf, dstf, y_v, t_v, by_v, bt_v,
             rowbuf, sem_bulk, sem_g, sem_s):
    wid = lax.axis_index("s") * _NC + lax.axis_index("c")
    base = wid * _RPW
    lane = lax.iota(jnp.int32, 16)

    copies = [
        pltpu.make_async_copy(
            bx_hbm.at[pl.ds(base + c * _BR, _BR)],
            obx_hbm.at[pl.ds(base + c * _BR, _BR)],
            sem_bulk,
        )
        for c in range(_RPW // _BR)
    ]
    for cp in copies:
        cp.start()

    pltpu.sync_copy(idx_hbm, idx_v)
    pltpu.sync_copy(y_hbm, y_v)
    pltpu.sync_copy(t_hbm, t_v)
    pltpu.sync_copy(by_hbm.at[pl.ds(base, _RPW)], by_v)
    pltpu.sync_copy(bt_hbm.at[pl.ds(base, _RPW)], bt_v)

    for s in range(_RPW // 16):
        last_v[pl.ds(s * 16, 16)] = jnp.full((16,), -1, jnp.int32)

    def win_step(v, _):
        iv = idx_v[pl.ds(v * 16, 16)]
        inr = (iv >= base) & (iv < base + _RPW)
        n_inr = jnp.sum(inr.astype(jnp.int32))

        @pl.when(n_inr > 0)
        def _():
            _, keep = plsc.scan_count(iv, mask=inr)
            plsc.store_scatter(last_v, [iv - base],
                               v * 16 + lane, mask=keep)
        return 0

    lax.fori_loop(0, _NV, win_step, 0)

    def cmp_step(s, off):
        lv = last_v[pl.ds(s * 16, 16)]
        m = lv >= 0
        cnt = jnp.sum(m.astype(jnp.int32))
        plsc.store_compressed(srcf.at[pl.ds(off, 16)], lv, mask=m)
        plsc.store_compressed(dstf.at[pl.ds(off, 16)], base + s * 16 + lane, mask=m)
        return off + cnt

    nw = lax.fori_loop(0, _RPW // 16, cmp_step, jnp.int32(0))

    @pl.when(nw > 0)
    def _():
        neg = jnp.full((16,), -(2**31), jnp.int32)
        zero16 = jnp.zeros((16,), jnp.int32)
        s0 = jnp.max(jnp.where(lane == 0, srcf[pl.ds(0, 16)], neg))
        d0 = jnp.max(jnp.where(lane == 0, dstf[pl.ds(0, 16)], neg))
        srcf[pl.ds(nw, 16)] = zero16 + s0
        dstf[pl.ds(nw, 16)] = zero16 + d0

    nch = (nw + _CH - 1) // _CH

    def lbl_step(w, _):
        sv = srcf[pl.ds(w * 16, 16)]
        dv = dstf[pl.ds(w * 16, 16)] - base
        plsc.store_scatter(by_v, [dv],
                           plsc.load_gather(y_v, [sv]))
        plsc.store_scatter(bt_v, [dv],
                           plsc.load_gather(t_v, [sv]))
        return 0

    lax.fori_loop(0, (nw + 15) // 16, lbl_step, 0)
    pltpu.sync_copy(by_v, oby_hbm.at[pl.ds(base, _RPW)])
    pltpu.sync_copy(bt_v, obt_hbm.at[pl.ds(base, _RPW)])

    for cp in copies:
        cp.wait()

    def mov_step(k, _):
        sv = srcf[pl.ds(k * _CH, _CH)]
        dv = dstf[pl.ds(k * _CH, _CH)]
        pltpu.async_copy(x_hbm.at[sv], rowbuf, sem_g).wait()
        pltpu.async_copy(rowbuf, obx_hbm.at[dv], sem_s).wait()
        return 0

    lax.fori_loop(0, nch, mov_step, 0)


@functools.partial(
    pl.kernel,
    out_type=[
        jax.ShapeDtypeStruct((_CAP, _D), jnp.float32),
        jax.ShapeDtypeStruct((_CAP,), jnp.int32),
        jax.ShapeDtypeStruct((_CAP,), jnp.int32),
    ],
    mesh=plsc.VectorSubcoreMesh(core_axis_name="c", subcore_axis_name="s"),
    compiler_params=pltpu.CompilerParams(needs_layout_passes=False),
    scratch_types=[
        pltpu.VMEM((_B,), jnp.int32),
        pltpu.VMEM((_RPW,), jnp.int32),
        pltpu.VMEM((_RPW + 16,), jnp.int32),
        pltpu.VMEM((_RPW + 16,), jnp.int32),
        pltpu.VMEM((_B,), jnp.int32),
        pltpu.VMEM((_B,), jnp.int32),
        pltpu.VMEM((_RPW,), jnp.int32),
        pltpu.VMEM((_RPW,), jnp.int32),
        pltpu.VMEM((_CH, _D), jnp.float32),
        pltpu.SemaphoreType.DMA,
        pltpu.SemaphoreType.DMA,
        pltpu.SemaphoreType.DMA,
    ],
)
def _sc_scatter(bx_hbm, by_hbm, bt_hbm, x_hbm, y_hbm, t_hbm, idx_hbm,
                obx_hbm, oby_hbm, obt_hbm, *scratch):
    _sc_body(bx_hbm, by_hbm, bt_hbm, x_hbm, y_hbm, t_hbm, idx_hbm,
             obx_hbm, oby_hbm, obt_hbm, *scratch)


def kernel(bx, by, bt, x, y, t, idx):
    obx, oby, obt = _sc_scatter(
        bx.reshape(_CAP, _D), by, bt, x.reshape(_B, _D), y, t, idx)
    return (obx.reshape(_CAP, 3, 32, 32), oby, obt)


# speedup vs baseline: 10.8749x; 10.8749x over previous
import functools

import jax
import jax.numpy as jnp
from jax import lax
from jax.experimental import pallas as pl
from jax.experimental.pallas import tpu as pltpu
from jax.experimental.pallas import tpu_sc as plsc

_CAP = 16384
_B = 4096
_D = 3 * 32 * 32

_NC = 2
_NS = 16
_NW = _NC * _NS
_RPW = _CAP // _NW
_NV = _B // 16
_CHB = 16
_CH = 16


def _sc_body(bx_hbm, by_hbm, bt_hbm, x_hbm, y_hbm, t_hbm, idx_hbm,
             obx_hbm, oby_hbm, obt_hbm,
             idx_v, last_v, srcf, dstf, y_v, t_v, by_v, bt_v,
             buf_a, buf_b, sem_g, sem_s):
    wid = lax.axis_index("s") * _NC + lax.axis_index("c")
    base = wid * _RPW
    lane = lax.iota(jnp.int32, 16)

    pltpu.sync_copy(idx_hbm, idx_v)
    pltpu.sync_copy(y_hbm, y_v)
    pltpu.sync_copy(t_hbm, t_v)
    pltpu.sync_copy(by_hbm.at[pl.ds(base, _RPW)], by_v)
    pltpu.sync_copy(bt_hbm.at[pl.ds(base, _RPW)], bt_v)

    for s in range(_RPW // 16):
        last_v[pl.ds(s * 16, 16)] = jnp.full((16,), -1, jnp.int32)

    def win_step(v, _):
        iv = idx_v[pl.ds(v * 16, 16)]
        inr = (iv >= base) & (iv < base + _RPW)
        n_inr = jnp.sum(inr.astype(jnp.int32))

        @pl.when(n_inr > 0)
        def _():
            _, keep = plsc.scan_count(iv, mask=inr)
            plsc.store_scatter(last_v, [iv - base],
                               v * 16 + lane, mask=keep)
        return 0

    lax.fori_loop(0, _NV, win_step, 0)

    def cmp_step(s, off):
        lv = last_v[pl.ds(s * 16, 16)]
        m = lv >= 0
        cnt = jnp.sum(m.astype(jnp.int32))
        plsc.store_compressed(srcf.at[pl.ds(off, 16)], lv, mask=m)
        plsc.store_compressed(dstf.at[pl.ds(off, 16)], base + s * 16 + lane, mask=m)
        return off + cnt

    nw = lax.fori_loop(0, _RPW // 16, cmp_step, jnp.int32(0))

    @pl.when(nw > 0)
    def _():
        neg = jnp.full((16,), -(2**31), jnp.int32)
        zero16 = jnp.zeros((16,), jnp.int32)
        s0 = jnp.max(jnp.where(lane == 0, srcf[pl.ds(0, 16)], neg))
        d0 = jnp.max(jnp.where(lane == 0, dstf[pl.ds(0, 16)], neg))
        srcf[pl.ds(nw, 16)] = zero16 + s0
        dstf[pl.ds(nw, 16)] = zero16 + d0

    nch = (nw + _CH - 1) // _CH

    def lbl_step(w, _):
        sv = srcf[pl.ds(w * 16, 16)]
        dv = dstf[pl.ds(w * 16, 16)] - base
        plsc.store_scatter(by_v, [dv],
                           plsc.load_gather(y_v, [sv]))
        plsc.store_scatter(bt_v, [dv],
                           plsc.load_gather(t_v, [sv]))
        return 0

    lax.fori_loop(0, (nw + 15) // 16, lbl_step, 0)
    pltpu.sync_copy(by_v, oby_hbm.at[pl.ds(base, _RPW)])
    pltpu.sync_copy(bt_v, obt_hbm.at[pl.ds(base, _RPW)])

    nbc = _RPW // _CHB
    bufs = (buf_a, buf_b)
    gops, sops = [], []
    for c in range(nbc):
        rows = pl.ds(base + c * _CHB, _CHB)
        gops.append(pltpu.make_async_copy(bx_hbm.at[rows], bufs[c % 2], sem_g))
        sops.append(pltpu.make_async_copy(bufs[c % 2], obx_hbm.at[rows], sem_s))
    gops[0].start()
    gops[1].start()
    for c in range(nbc):
        gops[c].wait()
        sops[c].start()
        sops[c].wait()
        if c + 2 < nbc:
            gops[c + 2].start()

    def mov_step(k, _):
        sv = srcf[pl.ds(k * _CH, _CH)]
        dv = dstf[pl.ds(k * _CH, _CH)]
        pltpu.async_copy(x_hbm.at[sv], buf_a, sem_g).wait()
        pltpu.async_copy(buf_a, obx_hbm.at[dv], sem_s).wait()
        return 0

    lax.fori_loop(0, nch, mov_step, 0)


@functools.partial(
    pl.kernel,
    out_type=[
        jax.ShapeDtypeStruct((_CAP, _D), jnp.float32),
        jax.ShapeDtypeStruct((_CAP,), jnp.int32),
        jax.ShapeDtypeStruct((_CAP,), jnp.int32),
    ],
    mesh=plsc.VectorSubcoreMesh(core_axis_name="c", subcore_axis_name="s"),
    compiler_params=pltpu.CompilerParams(needs_layout_passes=False),
    scratch_types=[
        pltpu.VMEM((_B,), jnp.int32),
        pltpu.VMEM((_RPW,), jnp.int32),
        pltpu.VMEM((_RPW + 16,), jnp.int32),
        pltpu.VMEM((_RPW + 16,), jnp.int32),
        pltpu.VMEM((_B,), jnp.int32),
        pltpu.VMEM((_B,), jnp.int32),
        pltpu.VMEM((_RPW,), jnp.int32),
        pltpu.VMEM((_RPW,), jnp.int32),
        pltpu.VMEM((_CHB, _D), jnp.float32),
        pltpu.VMEM((_CHB, _D), jnp.float32),
        pltpu.SemaphoreType.DMA,
        pltpu.SemaphoreType.DMA,
    ],
)
def _sc_scatter(bx_hbm, by_hbm, bt_hbm, x_hbm, y_hbm, t_hbm, idx_hbm,
                obx_hbm, oby_hbm, obt_hbm, *scratch):
    _sc_body(bx_hbm, by_hbm, bt_hbm, x_hbm, y_hbm, t_hbm, idx_hbm,
             obx_hbm, oby_hbm, obt_hbm, *scratch)


def kernel(bx, by, bt, x, y, t, idx):
    obx, oby, obt = _sc_scatter(
        bx.reshape(_CAP, _D), by, bt, x.reshape(_B, _D), y, t, idx)
    return (obx.reshape(_CAP, 3, 32, 32), oby, obt)


# speedup vs baseline: 14.1009x; 1.2967x over previous
import jax
import jax.numpy as jnp
from jax import lax
from jax.experimental import pallas as pl
from jax.experimental.pallas import tpu as pltpu
from jax.experimental.pallas import tpu_sc as plsc
from jax._src.pallas.mpmd import _mpmd_map

_CAP = 16384
_B = 4096
_D = 3 * 32 * 32

_NC = 2
_NS = 16
_NW = _NC * _NS
_RPW = _CAP // _NW
_NV = _B // 16
_CH = 16


def _sc_body(bx_hbm, by_hbm, bt_hbm, x_hbm, y_hbm, t_hbm, idx_hbm,
             obx_hbm, oby_hbm, obt_hbm,
             idx_v, last_v, srcf, dstf, y_v, t_v, by_v, bt_v,
             buf_a, buf_b, sem_g, sem_s):
    del bx_hbm
    wid = lax.axis_index("s") * _NC + lax.axis_index("c")
    base = wid * _RPW
    lane = lax.iota(jnp.int32, 16)

    pltpu.sync_copy(idx_hbm, idx_v)
    pltpu.sync_copy(y_hbm, y_v)
    pltpu.sync_copy(t_hbm, t_v)
    pltpu.sync_copy(by_hbm.at[pl.ds(base, _RPW)], by_v)
    pltpu.sync_copy(bt_hbm.at[pl.ds(base, _RPW)], bt_v)

    for s in range(_RPW // 16):
        last_v[pl.ds(s * 16, 16)] = jnp.full((16,), -1, jnp.int32)

    def win_step(v, _):
        iv = idx_v[pl.ds(v * 16, 16)]
        inr = (iv >= base) & (iv < base + _RPW)
        n_inr = jnp.sum(inr.astype(jnp.int32))

        @pl.when(n_inr > 0)
        def _():
            _, keep = plsc.scan_count(iv, mask=inr)
            plsc.store_scatter(last_v, [iv - base],
                               v * 16 + lane, mask=keep)
        return 0

    lax.fori_loop(0, _NV, win_step, 0)

    def cmp_step(s, off):
        lv = last_v[pl.ds(s * 16, 16)]
        m = lv >= 0
        cnt = jnp.sum(m.astype(jnp.int32))
        plsc.store_compressed(srcf.at[pl.ds(off, 16)], lv, mask=m)
        plsc.store_compressed(dstf.at[pl.ds(off, 16)], base + s * 16 + lane, mask=m)
        return off + cnt

    nw = lax.fori_loop(0, _RPW // 16, cmp_step, jnp.int32(0))

    @pl.when(nw > 0)
    def _():
        neg = jnp.full((16,), -(2**31), jnp.int32)
        zero16 = jnp.zeros((16,), jnp.int32)
        s0 = jnp.max(jnp.where(lane == 0, srcf[pl.ds(0, 16)], neg))
        d0 = jnp.max(jnp.where(lane == 0, dstf[pl.ds(0, 16)], neg))
        srcf[pl.ds(nw, 16)] = zero16 + s0
        dstf[pl.ds(nw, 16)] = zero16 + d0

    nch = (nw + _CH - 1) // _CH

    def lbl_step(w, _):
        sv = srcf[pl.ds(w * 16, 16)]
        dv = dstf[pl.ds(w * 16, 16)] - base
        plsc.store_scatter(by_v, [dv],
                           plsc.load_gather(y_v, [sv]))
        plsc.store_scatter(bt_v, [dv],
                           plsc.load_gather(t_v, [sv]))
        return 0

    lax.fori_loop(0, (nw + 15) // 16, lbl_step, 0)
    pltpu.sync_copy(by_v, oby_hbm.at[pl.ds(base, _RPW)])
    pltpu.sync_copy(bt_v, obt_hbm.at[pl.ds(base, _RPW)])

    def mov_pair(p, _):
        k0 = p * 2
        sv0 = srcf[pl.ds(k0 * _CH, _CH)]
        pltpu.make_async_copy(x_hbm.at[sv0], buf_a, sem_g).start()

        @pl.when(k0 + 1 < nch)
        def _():
            sv1 = srcf[pl.ds((k0 + 1) * _CH, _CH)]
            pltpu.make_async_copy(x_hbm.at[sv1], buf_b, sem_g).start()

        pltpu.make_async_copy(x_hbm.at[sv0], buf_a, sem_g).wait()
        dv0 = dstf[pl.ds(k0 * _CH, _CH)]
        pltpu.async_copy(buf_a, obx_hbm.at[dv0], sem_s).wait()

        @pl.when(k0 + 1 < nch)
        def _():
            sv1 = srcf[pl.ds((k0 + 1) * _CH, _CH)]
            pltpu.make_async_copy(x_hbm.at[sv1], buf_b, sem_g).wait()
            dv1 = dstf[pl.ds((k0 + 1) * _CH, _CH)]
            pltpu.async_copy(buf_b, obx_hbm.at[dv1], sem_s).wait()
        return 0

    lax.fori_loop(0, (nch + 1) // 2, mov_pair, 0)


_sc_scatter = _mpmd_map(
    [(plsc.VectorSubcoreMesh(core_axis_name="c", subcore_axis_name="s"),
      _sc_body)],
    [
        jax.ShapeDtypeStruct((_CAP, _D), jnp.float32),
        jax.ShapeDtypeStruct((_CAP,), jnp.int32),
        jax.ShapeDtypeStruct((_CAP,), jnp.int32),
    ],
    input_output_aliases={0: 0, 1: 1, 2: 2},
    scratch_types=[
        pltpu.VMEM((_B,), jnp.int32),
        pltpu.VMEM((_RPW,), jnp.int32),
        pltpu.VMEM((_RPW + 16,), jnp.int32),
        pltpu.VMEM((_RPW + 16,), jnp.int32),
        pltpu.VMEM((_B,), jnp.int32),
        pltpu.VMEM((_B,), jnp.int32),
        pltpu.VMEM((_RPW,), jnp.int32),
        pltpu.VMEM((_RPW,), jnp.int32),
        pltpu.VMEM((_CH, _D), jnp.float32),
        pltpu.VMEM((_CH, _D), jnp.float32),
        pltpu.SemaphoreType.DMA,
        pltpu.SemaphoreType.DMA,
    ],
    compiler_params=pltpu.CompilerParams(needs_layout_passes=False),
)


def kernel(bx, by, bt, x, y, t, idx):
    obx, oby, obt = _sc_scatter(
        bx.reshape(_CAP, _D), by, bt, x.reshape(_B, _D), y, t, idx)
    return (obx.reshape(_CAP, 3, 32, 32), oby, obt)
